# Initial kernel scaffold; baseline (speedup 1.0000x reference)
import jax
import jax.numpy as jnp
from jax.experimental import pallas as pl


def kernel(structure, sequence, t):
    raise NotImplementedError("write your pallas kernel here")


# speedup vs baseline: 1.9569x; 1.9569x over previous
import functools

import jax
import jax.numpy as jnp
import numpy as np
from jax.experimental import pallas as pl
from jax.experimental.pallas import tpu as pltpu

T = 500
STRUC_N = 517
SEQ_N = 33
STRUC_MASK = 516
SEQ_MASK = 32
EPS = 1e-10
_NEG = np.int32(-(2 ** 31))
_BIG = np.int32(2 ** 30)


def _threefry_bits(k1, k2, x1):
    ks0 = k1
    ks1 = k2
    ks2 = k1 ^ k2 ^ jnp.uint32(0x1BD11BDA)
    ks = (ks0, ks1, ks2)
    x0 = jnp.zeros_like(x1) + ks0
    x1 = x1 + ks1
    rot = (13, 15, 26, 6, 17, 29, 16, 24)
    rounds = (rot[0:4], rot[4:8], rot[0:4], rot[4:8], rot[0:4])
    for i, chunk in enumerate(rounds):
        for r in chunk:
            x0 = x0 + x1
            x1 = (x1 << jnp.uint32(r)) | (x1 >> jnp.uint32(32 - r))
            x1 = x0 ^ x1
        x0 = x0 + ks[(i + 1) % 3]
        x1 = x1 + ks[(i + 2) % 3] + jnp.uint32(i + 1)
    return x0 ^ x1


def _sample_body(key_ref, x0_ref, out_ref, *, n_cls, n_pad, mask_idx, lanes):
    p = pl.program_id(0)
    k1 = jax.lax.bitcast_convert_type(key_ref[0], jnp.uint32)
    k2 = jax.lax.bitcast_convert_type(key_ref[1], jnp.uint32)
    c = jax.lax.broadcasted_iota(jnp.int32, (n_pad, lanes), 0)
    lane = jax.lax.broadcasted_iota(jnp.int32, (n_pad, lanes), 1)
    row = p * lanes + lane
    i = (row * n_cls + c).astype(jnp.uint32)
    bits = _threefry_bits(k1, k2, i)
    biased = jax.lax.bitcast_convert_type(bits ^ jnp.uint32(0x80000000),
                                          jnp.int32)
    x0 = x0_ref[0]
    is_x0 = c == x0
    is_mask = c == mask_idx
    excl = is_x0 | is_mask | (c >= n_cls)
    b_eps = jnp.where(excl, _NEG, biased)
    eps_max = jnp.max(b_eps, axis=0, keepdims=True)
    idx_eps = jnp.min(jnp.where(b_eps == eps_max, c, _BIG), axis=0,
                      keepdims=True)
    b_x0 = jnp.max(jnp.where(is_x0, biased, _NEG), axis=0, keepdims=True)
    b_mask = jnp.max(jnp.where(is_mask, biased, _NEG), axis=0, keepdims=True)
    out_ref[0, 0:1, :] = b_x0
    out_ref[0, 1:2, :] = b_mask
    out_ref[0, 2:3, :] = eps_max
    out_ref[0, 3:4, :] = idx_eps
    out_ref[0, 4:8, :] = jnp.zeros((4, lanes), jnp.int32)


def _candidates(x_flat, key_data, n_cls, n_pad, mask_idx, lanes):
    rows = x_flat.shape[0]
    grid = rows // lanes
    x_in = x_flat.reshape(grid, 1, lanes)
    body = functools.partial(_sample_body, n_cls=n_cls, n_pad=n_pad,
                             mask_idx=mask_idx, lanes=lanes)
    out = pl.pallas_call(
        body,
        grid=(grid,),
        in_specs=[
            pl.BlockSpec(memory_space=pltpu.SMEM),
            pl.BlockSpec((1, 1, lanes), lambda p: (p, 0, 0)),
        ],
        out_specs=pl.BlockSpec((1, 8, lanes), lambda p: (p, 0, 0)),
        out_shape=jax.ShapeDtypeStruct((grid, 8, lanes), jnp.int32),
        compiler_params=pltpu.CompilerParams(
            dimension_semantics=("parallel",)),
    )(key_data.astype(jnp.int32), x_in)
    unbias = lambda b: jax.lax.bitcast_convert_type(b, jnp.uint32) ^ jnp.uint32(
        0x80000000)
    b_x0 = unbias(out[:, 0, :].reshape(rows))
    b_mask = unbias(out[:, 1, :].reshape(rows))
    b_eps = unbias(out[:, 2, :].reshape(rows))
    i_eps = out[:, 3, :].reshape(rows)
    return b_x0, b_mask, b_eps, i_eps


def _gumbel_from_bits(bits):
    tiny = jnp.float32(jnp.finfo(jnp.float32).tiny)
    fb = (bits >> jnp.uint32(9)) | jnp.uint32(0x3F800000)
    floats = jax.lax.bitcast_convert_type(fb, jnp.float32) - jnp.float32(1.0)
    u = jnp.maximum(tiny, floats * (jnp.float32(1.0) - tiny) + tiny)
    return -jnp.log(-jnp.log(u))


def _finish(b_x0, b_mask, b_eps, i_eps, x_flat, a_flat, mask_idx):
    eq = x_flat == mask_idx
    one_minus_a = jnp.float32(1.0) - a_flat
    p_x0 = a_flat + jnp.where(eq, one_minus_a, jnp.float32(0.0))
    p_m = jnp.where(eq, a_flat + one_minus_a, one_minus_a)
    v1 = _gumbel_from_bits(b_x0) + jnp.log(p_x0 + EPS)
    v2 = _gumbel_from_bits(b_mask) + jnp.log(p_m + EPS)
    v3 = _gumbel_from_bits(b_eps) + jnp.log(jnp.float32(0.0) + EPS)
    i1 = x_flat
    i2 = jnp.full_like(x_flat, mask_idx)
    best_v, best_i = v1, i1
    upd = (v2 > best_v) | ((v2 == best_v) & (i2 < best_i))
    best_v = jnp.where(upd, v2, best_v)
    best_i = jnp.where(upd, i2, best_i)
    upd = (v3 > best_v) | ((v3 == best_v) & (i_eps < best_i))
    best_i = jnp.where(upd, i_eps, best_i)
    return best_i


def kernel(structure, sequence, t):
    t_idx = jnp.arange(T + 1, dtype=jnp.float32)
    beta = 1.0 / (T - t_idx + 1.0)
    alpha = jnp.cumprod(1.0 - beta)
    key = jax.random.key(42)
    ks, kq = jax.random.split(key)
    kd_s = jax.random.key_data(ks)
    kd_q = jax.random.key_data(kq)
    B, L = structure.shape
    a_flat = jnp.repeat(alpha[t], L)
    outs = []
    for x, kd, n_cls, n_pad, mask_idx, lanes in (
            (structure, kd_s, STRUC_N, 520, STRUC_MASK, 128),
            (sequence, kd_q, SEQ_N, 40, SEQ_MASK, 512)):
        x_flat = x.reshape(-1).astype(jnp.int32)
        cands = _candidates(x_flat, kd, n_cls, n_pad, mask_idx, lanes)
        tok = _finish(*cands, x_flat, a_flat, mask_idx)
        outs.append(tok.reshape(B, L))
    return outs[0], outs[1], t
